# baseline (speedup 1.0000x reference)
import functools

import jax
import jax.numpy as jnp
from jax import lax
from jax.experimental import pallas as pl
from jax.experimental.pallas import tpu as pltpu
from jax.experimental.pallas import tpu_sc as plsc

D = 64
SCALE = 8.0
NC = 2
NS = 16
L = 16
NW = NC * NS
CS = 128
NBUF = 4


@functools.lru_cache(maxsize=None)
def _make_emb(CH):
    G = CH // NBUF
    mesh = plsc.VectorSubcoreMesh(core_axis_name="c", subcore_axis_name="s")

    @functools.partial(
        pl.kernel,
        out_type=jax.ShapeDtypeStruct((NW, CH, CS, D), jnp.float32),
        mesh=mesh,
        compiler_params=pltpu.CompilerParams(use_tc_tiling_on_sc=False),
        scratch_types=[
            pltpu.VMEM((CH, CS), jnp.int32),
            pltpu.VMEM((NBUF, CS, D), jnp.float32),
            pltpu.VMEM((NBUF, CS, D), jnp.float32),
            pltpu.SemaphoreType.DMA((NBUF,)),
            pltpu.SemaphoreType.DMA((NBUF,)),
        ],
    )
    def emb(x_hbm, lut_hbm, out_hbm, idx_v, rows_g, rows_o, gsem, wsem):
        wid = lax.axis_index("s") * NC + lax.axis_index("c")
        pltpu.sync_copy(x_hbm.at[wid], idx_v)

        def fire_gather(j, b):
            pltpu.async_copy(lut_hbm.at[idx_v.at[j]], rows_g.at[b], gsem.at[b])

        def wait_gather(b):
            pltpu.make_async_copy(
                out_hbm.at[0, 0], rows_g.at[b], gsem.at[b]).wait()

        def fire_write(j, b):
            pltpu.async_copy(rows_o.at[b], out_hbm.at[wid, j], wsem.at[b])

        def wait_write(b):
            pltpu.make_async_copy(
                out_hbm.at[0, 0], rows_o.at[b], wsem.at[b]).wait()

        def scale(b):
            def row(r, carry):
                for c in range(D // L):
                    sl = pl.ds(c * L, L)
                    rows_o[b, r, sl] = rows_g[b, r, sl] * SCALE
                return carry
            lax.fori_loop(0, CS, row, 0)

        def chunk_step(g, b, first, refire):
            j = g * NBUF + b
            wait_gather(b)
            if not first:
                wait_write(b)
            scale(b)
            if refire:
                fire_gather(j + NBUF, b)
            fire_write(j, b)

        for b in range(NBUF):
            fire_gather(b, b)
        for b in range(NBUF):
            chunk_step(0, b, True, True)

        def body(g, carry):
            for b in range(NBUF):
                chunk_step(g, b, False, True)
            return carry

        lax.fori_loop(1, G - 1, body, 0)

        for b in range(NBUF):
            chunk_step(G - 1, b, False, False)
        for b in range(NBUF):
            wait_write(b)

    return emb


def kernel(x, lut):
    S, Bt = x.shape
    B = S * Bt
    per_w = B // NW
    CH = per_w // CS
    xr = x.astype(jnp.int32).reshape(NW, CH, CS)
    out = _make_emb(CH)(xr, lut)
    return out.reshape(S, Bt, D)
